# baseline (speedup 1.0000x reference)
import functools
import numpy as np
import jax
import jax.numpy as jnp
from jax import lax
from jax.experimental import pallas as pl
from jax.experimental.pallas import tpu as pltpu
from jax.experimental.pallas import tpu_sc as plsc

SEG = 8
NVAR = 100
H = 16
D = 1024
G = 10
K = 10
DH = D // H
R = H * G
SCALE = float((D / H) ** -0.5)

NW = 32
CB = 8


def _q_kernel(cl_ref, wq_ref, bq_ref, qblk_ref):
    qq = jnp.dot(cl_ref[0], wq_ref[...], preferred_element_type=jnp.float32)
    qq = qq + bq_ref[...]
    q3 = jnp.broadcast_to(qq[None, :, :], (H, G, D))
    h_iota = lax.broadcasted_iota(jnp.int32, (H, G, D), 0)
    d_iota = lax.broadcasted_iota(jnp.int32, (H, G, D), 2)
    qblk_ref[0] = jnp.where(d_iota // DH == h_iota, q3, 0.0).reshape(R, D)


def _tc1_kernel(x_ref, qblk_ref, wk_ref, bk_ref, wv_ref, bv_ref, p_ref,
                v_ref, attn_ref, idx_ref, sidx_ref):
    bs_i = pl.program_id(1) * SEG + pl.program_id(0)
    a = x_ref[0]
    a = jnp.log(jnp.maximum(a, 0.0) + 1.0)
    kb = jnp.dot(a, wk_ref[...], preferred_element_type=jnp.float32) + bk_ref[...]
    vb = jnp.dot(a, wv_ref[...], preferred_element_type=jnp.float32) + bv_ref[...]
    for j in range(8):
        v_ref[pl.ds(j * NVAR, NVAR)] = vb[:, j * 128:(j + 1) * 128]
    scores = lax.dot_general(
        qblk_ref[0], kb, (((1,), (1,)), ((), ())),
        preferred_element_type=jnp.float32) * SCALE

    lane = lax.broadcasted_iota(jnp.int32, (R, NVAR), 1)
    s = scores
    vals, idxs = [], []
    for _ in range(K):
        m = jnp.max(s, axis=1, keepdims=True)
        cand = jnp.where(s == m, lane, jnp.int32(NVAR))
        j = jnp.min(cand, axis=1, keepdims=True)
        vals.append(m)
        idxs.append(j)
        s = jnp.where(lane == j, -jnp.inf, s)
    topv = jnp.concatenate(vals, axis=1)
    topi = jnp.concatenate(idxs, axis=1)

    e = jnp.exp(topv - topv[:, 0:1])
    attn_ref[0] = e / jnp.sum(e, axis=1, keepdims=True)
    idx_ref[0] = topi
    hrow = lax.broadcasted_iota(jnp.int32, (R, K), 0) // G
    val = (bs_i * 8 + hrow // 2) * NVAR + topi
    sp = jnp.dot(p_ref[...], val.astype(jnp.float32),
                 preferred_element_type=jnp.float32,
                 precision=lax.Precision.HIGHEST)
    sidx_ref[0] = sp.astype(jnp.int32)


def _sc_kernel(sidx_hbm, v_hbm, wg_hbm, bg_hbm, xo_hbm,
               sidx_all, rows0, rows1, outb, wg_bc, bg_bc, sem0, sem1):
    c = lax.axis_index("c")
    s = lax.axis_index("s")
    wid = s * 2 + c
    pltpu.sync_copy(wg_hbm, wg_bc)
    pltpu.sync_copy(bg_hbm, bg_bc)
    bs = xo_hbm.shape[0] // R
    per_w = (bs * R) // NW
    base = wid * per_w
    nch = per_w // CB
    pch = R // CB
    pltpu.sync_copy(sidx_hbm.at[pl.ds(wid * nch, nch)], sidx_all)

    bufs = (rows0, rows1)
    sems = (sem0, sem1)
    pltpu.async_copy(v_hbm.at[sidx_all.at[0]], rows0, sem0)

    def chunk(t, ph):
        tn = jnp.minimum(t + 1, nch - 1)
        pltpu.async_copy(
            v_hbm.at[sidx_all.at[tn]], bufs[1 - ph], sems[1 - ph])
        pltpu.make_async_copy(
            v_hbm.at[sidx_all.at[t]], bufs[ph], sems[ph]).wait()
        rows = bufs[ph]
        for i in range(CB):
            ridx = base + t * CB + i
            dl = lax.rem(ridx, R)
            g = lax.div(dl, H)
            h = lax.rem(dl, H)
            col0 = lax.rem(h, 2) * DH
            wvecs = [wg_bc[g * K + r] for r in range(K)]
            accs = []
            for q in range(4):
                acc = jnp.zeros((16,), jnp.float32)
                for r in range(K):
                    acc = acc + rows[i * K + r,
                                     pl.ds(col0 + q * 16, 16)] * wvecs[r]
                accs.append(acc + bg_bc[g])
            mx4 = jnp.maximum(jnp.maximum(accs[0], accs[1]),
                              jnp.maximum(accs[2], accs[3]))
            mn4 = jnp.minimum(jnp.minimum(accs[0], accs[1]),
                              jnp.minimum(accs[2], accs[3]))
            mx = lax.reduce_max(mx4, (0,))
            mn = lax.reduce_min(mn4, (0,))
            denom = jnp.maximum(mx - mn, 1e-6)
            lr = lax.rem(t, pch) * CB + i
            for q in range(4):
                outb[lr, pl.ds(q * 16, 16)] = jnp.exp((accs[q] - mn) / denom)
        @pl.when(lax.rem(t, pch) == pch - 1)
        def _():
            bs_i = lax.div(base + t * CB, R)
            pltpu.sync_copy(outb, xo_hbm.at[pl.ds(bs_i * R, R)])

    def body(u, carry):
        chunk(2 * u, 0)
        chunk(2 * u + 1, 1)
        return carry

    lax.fori_loop(0, nch // 2, body, 0)
    pltpu.make_async_copy(
        v_hbm.at[sidx_all.at[nch - 1]], rows0, sem0).wait()


def _out_kernel(xo_ref, wo_ref, bo_ref, out_ref):
    out_ref[...] = jnp.dot(xo_ref[...], wo_ref[...],
                           preferred_element_type=jnp.float32) + bo_ref[...]


@jax.jit
def _run(x, cluster, Wq, bq, Wk, bk, Wv, bv, Wg, bg, Wo, bo):
    bs = x.shape[0]
    nb = bs // SEG

    qblk = pl.pallas_call(
        _q_kernel,
        grid=(SEG,),
        in_specs=[
            pl.BlockSpec((1, G, D), lambda s: (s, 0, 0)),
            pl.BlockSpec((D, D), lambda s: (0, 0)),
            pl.BlockSpec((1, D), lambda s: (0, 0)),
        ],
        out_specs=pl.BlockSpec((1, R, D), lambda s: (s, 0, 0)),
        out_shape=jax.ShapeDtypeStruct((SEG, R, D), jnp.float32),
    )(cluster, Wq, bq.reshape(1, D))

    dl = np.arange(R)
    pmat = np.zeros((R, R), np.float32)
    pmat[dl, (dl % H) * G + dl // H] = 1.0
    perm = jnp.asarray(pmat)
    wg_bc = jnp.broadcast_to(Wg.reshape(G * K, 1), (G * K, 16))
    bg_bc = jnp.broadcast_to(bg.reshape(G, 1), (G, 16))

    def run_chunk(xc):
        cbs = xc.shape[0]
        cnb = cbs // SEG
        v, attn_t, idx_t, sidx_t = pl.pallas_call(
            _tc1_kernel,
            grid=(SEG, cnb),
            in_specs=[
                pl.BlockSpec((1, NVAR, D), lambda s, b: (b * SEG + s, 0, 0)),
                pl.BlockSpec((1, R, D), lambda s, b: (s, 0, 0)),
                pl.BlockSpec((D, D), lambda s, b: (0, 0)),
                pl.BlockSpec((1, D), lambda s, b: (0, 0)),
                pl.BlockSpec((D, D), lambda s, b: (0, 0)),
                pl.BlockSpec((1, D), lambda s, b: (0, 0)),
                pl.BlockSpec((R, R), lambda s, b: (0, 0)),
            ],
            out_specs=[
                pl.BlockSpec((NVAR * 8, 128), lambda s, b: (b * SEG + s, 0)),
                pl.BlockSpec((1, R, K), lambda s, b: (b * SEG + s, 0, 0)),
                pl.BlockSpec((1, R, K), lambda s, b: (b * SEG + s, 0, 0)),
                pl.BlockSpec((1, R, K), lambda s, b: (b * SEG + s, 0, 0)),
            ],
            out_shape=[
                jax.ShapeDtypeStruct((cbs * NVAR * 8, 128), jnp.float32),
                jax.ShapeDtypeStruct((cbs, R, K), jnp.float32),
                jax.ShapeDtypeStruct((cbs, R, K), jnp.int32),
                jax.ShapeDtypeStruct((cbs, R, K), jnp.int32),
            ],
        )(xc, qblk, Wk, bk.reshape(1, D), Wv, bv.reshape(1, D), perm)

        sc = pl.kernel(
            _sc_kernel,
            out_type=jax.ShapeDtypeStruct((cbs * R, DH), jnp.float32),
            mesh=plsc.VectorSubcoreMesh(core_axis_name="c",
                                        subcore_axis_name="s",
                                        num_cores=2, num_subcores=16),
            compiler_params=pltpu.CompilerParams(needs_layout_passes=False),
            scratch_types=[
                pltpu.VMEM(((cbs * R // NW) // CB, CB * K), jnp.int32),
                pltpu.VMEM((CB * K, 128), jnp.float32),
                pltpu.VMEM((CB * K, 128), jnp.float32),
                pltpu.VMEM((R, DH), jnp.float32),
                pltpu.VMEM((G * K, 16), jnp.float32),
                pltpu.VMEM((G, 16), jnp.float32),
                pltpu.SemaphoreType.DMA,
                pltpu.SemaphoreType.DMA,
            ],
        )
        xo = sc(sidx_t.reshape(cbs * R * K // (CB * K), CB * K),
                v, wg_bc, bg_bc)

        xo_mat = xo.reshape(cbs * G, D)
        rb = 256 if (cbs * G) % 256 == 0 else cbs * G
        out = pl.pallas_call(
            _out_kernel,
            grid=((cbs * G) // rb,),
            in_specs=[
                pl.BlockSpec((rb, D), lambda i: (i, 0)),
                pl.BlockSpec((D, D), lambda i: (0, 0)),
                pl.BlockSpec((1, D), lambda i: (0, 0)),
            ],
            out_specs=pl.BlockSpec((rb, D), lambda i: (i, 0)),
            out_shape=jax.ShapeDtypeStruct((cbs * G, D), jnp.float32),
        )(xo_mat, Wo, bo.reshape(1, D))
        return out.reshape(cbs, G, D), attn_t, idx_t

    nchunks = 2 if bs % 256 == 0 else 1
    if nchunks == 2:
        h = bs // 2
        o1, a1, i1 = run_chunk(x[:h])
        o2, a2, i2 = run_chunk(x[h:])
        out = jnp.concatenate([o1, o2], axis=0)
        attn_t = jnp.concatenate([a1, a2], axis=0)
        idx_t = jnp.concatenate([i1, i2], axis=0)
    else:
        out, attn_t, idx_t = run_chunk(x)

    return (out,
            attn_t.reshape(bs, H, G, K),
            idx_t.reshape(bs, H, G, K))


def kernel(x, cluster, Wq, bq, Wk, bk, Wv, bv, Wg, bg, Wo, bo):
    return _run(x, cluster, Wq, bq, Wk, bk, Wv, bv, Wg, bg, Wo, bo)


# speedup vs baseline: 1.3272x; 1.3272x over previous
import functools
import numpy as np
import jax
import jax.numpy as jnp
from jax import lax
from jax.experimental import pallas as pl
from jax.experimental.pallas import tpu as pltpu
from jax.experimental.pallas import tpu_sc as plsc

SEG = 8
NVAR = 100
H = 16
D = 1024
G = 10
K = 10
DH = D // H
R = H * G
SCALE = float((D / H) ** -0.5)

NW = 32
CB = 8


def _q_kernel(cl_ref, wq_ref, bq_ref, qblk_ref):
    qq = jnp.dot(cl_ref[0], wq_ref[...], preferred_element_type=jnp.float32)
    qq = qq + bq_ref[...]
    q3 = jnp.broadcast_to(qq[None, :, :], (H, G, D))
    h_iota = lax.broadcasted_iota(jnp.int32, (H, G, D), 0)
    d_iota = lax.broadcasted_iota(jnp.int32, (H, G, D), 2)
    qblk_ref[0] = jnp.where(d_iota // DH == h_iota, q3, 0.0).reshape(R, D)


def _tc1_kernel(x_ref, qblk_ref, wk_ref, bk_ref, wv_ref, bv_ref, p_ref,
                v_ref, attn_ref, idx_ref, sidx_ref):
    bsA = 2 * pl.program_id(1) * SEG + pl.program_id(0)
    a2 = x_ref[:, 0].reshape(2 * NVAR, D)
    a2 = jnp.log(jnp.maximum(a2, 0.0) + 1.0)
    kb = jnp.dot(a2, wk_ref[...], preferred_element_type=jnp.float32) + bk_ref[...]
    vb = jnp.dot(a2, wv_ref[...], preferred_element_type=jnp.float32) + bv_ref[...]
    for b in range(2):
        for j in range(8):
            v_ref[b, 0, pl.ds(j * NVAR, NVAR)] = (
                vb[b * NVAR:(b + 1) * NVAR, j * 128:(j + 1) * 128])
    qblk = qblk_ref[0]
    sc_halves = [
        lax.dot_general(
            qblk, kb[b * NVAR:(b + 1) * NVAR], (((1,), (1,)), ((), ())),
            preferred_element_type=jnp.float32) * SCALE
        for b in range(2)]
    s = jnp.concatenate(sc_halves, axis=0)

    lane = lax.broadcasted_iota(jnp.int32, (2 * R, NVAR), 1)
    vals, idxs = [], []
    for _ in range(K):
        m = jnp.max(s, axis=1, keepdims=True)
        cand = jnp.where(s == m, lane, jnp.int32(NVAR))
        j = jnp.min(cand, axis=1, keepdims=True)
        vals.append(m)
        idxs.append(j)
        s = jnp.where(lane == j, -jnp.inf, s)
    topv = jnp.concatenate(vals, axis=1)
    topi = jnp.concatenate(idxs, axis=1)

    e = jnp.exp(topv - topv[:, 0:1])
    attn = e / jnp.sum(e, axis=1, keepdims=True)
    hrow2 = lax.broadcasted_iota(jnp.int32, (2 * R, K), 0) % R // G
    bsvec = bsA + SEG * (lax.broadcasted_iota(jnp.int32, (2 * R, K), 0) // R)
    val = bsvec * (NVAR * 8) + (hrow2 // 2) * NVAR + topi
    for b in range(2):
        attn_ref[b, 0] = attn[b * R:(b + 1) * R]
        idx_ref[b, 0] = topi[b * R:(b + 1) * R]
        sp = jnp.dot(p_ref[...],
                     val[b * R:(b + 1) * R].astype(jnp.float32),
                     preferred_element_type=jnp.float32,
                     precision=lax.Precision.HIGHEST)
        sidx_ref[b, 0] = sp.astype(jnp.int32)


def _sc_kernel(sidx_hbm, v_hbm, wg_hbm, bg_hbm, xo_hbm,
               sidx_all, rows0, rows1, outb, wg_bc, bg_bc, sem0, sem1):
    c = lax.axis_index("c")
    s = lax.axis_index("s")
    wid = s * 2 + c
    pltpu.sync_copy(wg_hbm, wg_bc)
    pltpu.sync_copy(bg_hbm, bg_bc)
    bs = xo_hbm.shape[0] // R
    per_w = (bs * R) // NW
    base = wid * per_w
    nch = per_w // CB
    pch = R // CB
    pltpu.sync_copy(sidx_hbm.at[pl.ds(wid * nch, nch)], sidx_all)

    bufs = (rows0, rows1)
    sems = (sem0, sem1)
    pltpu.async_copy(v_hbm.at[sidx_all.at[0]], rows0, sem0)

    def chunk(t, ph):
        tn = jnp.minimum(t + 1, nch - 1)
        pltpu.async_copy(
            v_hbm.at[sidx_all.at[tn]], bufs[1 - ph], sems[1 - ph])
        pltpu.make_async_copy(
            v_hbm.at[sidx_all.at[t]], bufs[ph], sems[ph]).wait()
        rows = bufs[ph]
        for i in range(CB):
            ridx = base + t * CB + i
            dl = lax.rem(ridx, R)
            g = lax.div(dl, H)
            h = lax.rem(dl, H)
            col0 = lax.rem(h, 2) * DH
            wvecs = [wg_bc[g * K + r] for r in range(K)]
            accs = []
            for q in range(4):
                acc = jnp.zeros((16,), jnp.float32)
                for r in range(K):
                    acc = acc + rows[i * K + r,
                                     pl.ds(col0 + q * 16, 16)] * wvecs[r]
                accs.append(acc + bg_bc[g])
            mx4 = jnp.maximum(jnp.maximum(accs[0], accs[1]),
                              jnp.maximum(accs[2], accs[3]))
            mn4 = jnp.minimum(jnp.minimum(accs[0], accs[1]),
                              jnp.minimum(accs[2], accs[3]))
            mx = lax.reduce_max(mx4, (0,))
            mn = lax.reduce_min(mn4, (0,))
            denom = jnp.maximum(mx - mn, 1e-6)
            lr = lax.rem(t, pch) * CB + i
            for q in range(4):
                outb[lr, pl.ds(q * 16, 16)] = jnp.exp((accs[q] - mn) / denom)
        @pl.when(lax.rem(t, pch) == pch - 1)
        def _():
            bs_i = lax.div(base + t * CB, R)
            pltpu.sync_copy(outb, xo_hbm.at[pl.ds(bs_i * R, R)])

    def body(u, carry):
        chunk(2 * u, 0)
        chunk(2 * u + 1, 1)
        return carry

    lax.fori_loop(0, nch // 2, body, 0)
    pltpu.make_async_copy(
        v_hbm.at[sidx_all.at[nch - 1]], rows0, sem0).wait()


def _out_kernel(xo_ref, wo_ref, bo_ref, out_ref):
    out_ref[...] = jnp.dot(xo_ref[...], wo_ref[...],
                           preferred_element_type=jnp.float32) + bo_ref[...]


@jax.jit
def _run(x, cluster, Wq, bq, Wk, bk, Wv, bv, Wg, bg, Wo, bo):
    bs = x.shape[0]
    nb = bs // SEG

    qblk = pl.pallas_call(
        _q_kernel,
        grid=(SEG,),
        in_specs=[
            pl.BlockSpec((1, G, D), lambda s: (s, 0, 0)),
            pl.BlockSpec((D, D), lambda s: (0, 0)),
            pl.BlockSpec((1, D), lambda s: (0, 0)),
        ],
        out_specs=pl.BlockSpec((1, R, D), lambda s: (s, 0, 0)),
        out_shape=jax.ShapeDtypeStruct((SEG, R, D), jnp.float32),
    )(cluster, Wq, bq.reshape(1, D))

    dl = np.arange(R)
    pmat = np.zeros((R, R), np.float32)
    pmat[dl, (dl % H) * G + dl // H] = 1.0
    perm = jnp.asarray(pmat)
    wg_bc = jnp.broadcast_to(Wg.reshape(G * K, 1), (G * K, 16))
    bg_bc = jnp.broadcast_to(bg.reshape(G, 1), (G, 16))

    def run_chunk(xc):
        cbs = xc.shape[0]
        cnb = cbs // SEG
        x4 = xc.reshape(cnb, SEG, NVAR, D)
        v4, attn4, idx4, sidx4 = pl.pallas_call(
            _tc1_kernel,
            grid=(SEG, cnb // 2),
            in_specs=[
                pl.BlockSpec((2, 1, NVAR, D), lambda s, b: (b, s, 0, 0)),
                pl.BlockSpec((1, R, D), lambda s, b: (s, 0, 0)),
                pl.BlockSpec((D, D), lambda s, b: (0, 0)),
                pl.BlockSpec((1, D), lambda s, b: (0, 0)),
                pl.BlockSpec((D, D), lambda s, b: (0, 0)),
                pl.BlockSpec((1, D), lambda s, b: (0, 0)),
                pl.BlockSpec((R, R), lambda s, b: (0, 0)),
            ],
            out_specs=[
                pl.BlockSpec((2, 1, NVAR * 8, 128), lambda s, b: (b, s, 0, 0)),
                pl.BlockSpec((2, 1, R, K), lambda s, b: (b, s, 0, 0)),
                pl.BlockSpec((2, 1, R, K), lambda s, b: (b, s, 0, 0)),
                pl.BlockSpec((2, 1, R, K), lambda s, b: (b, s, 0, 0)),
            ],
            out_shape=[
                jax.ShapeDtypeStruct((cnb, SEG, NVAR * 8, 128), jnp.float32),
                jax.ShapeDtypeStruct((cnb, SEG, R, K), jnp.float32),
                jax.ShapeDtypeStruct((cnb, SEG, R, K), jnp.int32),
                jax.ShapeDtypeStruct((cnb, SEG, R, K), jnp.int32),
            ],
        )(x4, qblk, Wk, bk.reshape(1, D), Wv, bv.reshape(1, D), perm)
        v = v4.reshape(cbs * NVAR * 8, 128)
        attn_t = attn4.reshape(cbs, R, K)
        idx_t = idx4.reshape(cbs, R, K)
        sidx_t = sidx4.reshape(cbs, R, K)

        sc = pl.kernel(
            _sc_kernel,
            out_type=jax.ShapeDtypeStruct((cbs * R, DH), jnp.float32),
            mesh=plsc.VectorSubcoreMesh(core_axis_name="c",
                                        subcore_axis_name="s",
                                        num_cores=2, num_subcores=16),
            compiler_params=pltpu.CompilerParams(needs_layout_passes=False),
            scratch_types=[
                pltpu.VMEM(((cbs * R // NW) // CB, CB * K), jnp.int32),
                pltpu.VMEM((CB * K, 128), jnp.float32),
                pltpu.VMEM((CB * K, 128), jnp.float32),
                pltpu.VMEM((R, DH), jnp.float32),
                pltpu.VMEM((G * K, 16), jnp.float32),
                pltpu.VMEM((G, 16), jnp.float32),
                pltpu.SemaphoreType.DMA,
                pltpu.SemaphoreType.DMA,
            ],
        )
        xo = sc(sidx_t.reshape(cbs * R * K // (CB * K), CB * K),
                v, wg_bc, bg_bc)

        xo_mat = xo.reshape(cbs * G, D)
        rb = 256 if (cbs * G) % 256 == 0 else cbs * G
        out = pl.pallas_call(
            _out_kernel,
            grid=((cbs * G) // rb,),
            in_specs=[
                pl.BlockSpec((rb, D), lambda i: (i, 0)),
                pl.BlockSpec((D, D), lambda i: (0, 0)),
                pl.BlockSpec((1, D), lambda i: (0, 0)),
            ],
            out_specs=pl.BlockSpec((rb, D), lambda i: (i, 0)),
            out_shape=jax.ShapeDtypeStruct((cbs * G, D), jnp.float32),
        )(xo_mat, Wo, bo.reshape(1, D))
        return out.reshape(cbs, G, D), attn_t, idx_t

    nchunks = 2 if bs % 256 == 0 else 1
    if nchunks == 2:
        h = bs // 2
        o1, a1, i1 = run_chunk(x[:h])
        o2, a2, i2 = run_chunk(x[h:])
        out = jnp.concatenate([o1, o2], axis=0)
        attn_t = jnp.concatenate([a1, a2], axis=0)
        idx_t = jnp.concatenate([i1, i2], axis=0)
    else:
        out, attn_t, idx_t = run_chunk(x)

    return (out,
            attn_t.reshape(bs, H, G, K),
            idx_t.reshape(bs, H, G, K))


def kernel(x, cluster, Wq, bq, Wk, bk, Wv, bv, Wg, bg, Wo, bo):
    return _run(x, cluster, Wq, bq, Wk, bk, Wv, bv, Wg, bg, Wo, bo)


# speedup vs baseline: 1.5535x; 1.1705x over previous
import functools
import numpy as np
import jax
import jax.numpy as jnp
from jax import lax
from jax.experimental import pallas as pl
from jax.experimental.pallas import tpu as pltpu
from jax.experimental.pallas import tpu_sc as plsc

SEG = 8
NVAR = 100
H = 16
D = 1024
G = 10
K = 10
DH = D // H
R = H * G
SCALE = float((D / H) ** -0.5)

NW = 32
CB = 8
PB = 4


def _q_kernel(cl_ref, wq_ref, bq_ref, qblk_ref):
    qq = jnp.dot(cl_ref[0], wq_ref[...], preferred_element_type=jnp.float32)
    qq = qq + bq_ref[...]
    q3 = jnp.broadcast_to(qq[None, :, :], (H, G, D))
    h_iota = lax.broadcasted_iota(jnp.int32, (H, G, D), 0)
    d_iota = lax.broadcasted_iota(jnp.int32, (H, G, D), 2)
    qblk_ref[0] = jnp.where(d_iota // DH == h_iota, q3, 0.0).reshape(R, D)


def _tc1_kernel(x_ref, qblk_ref, wk_ref, bk_ref, wv_ref, bv_ref, p_ref,
                v_ref, attn_ref, idx_ref, sidx_ref):
    bsA = PB * pl.program_id(1) * SEG + pl.program_id(0)
    a2 = x_ref[:, 0].reshape(PB * NVAR, D)
    a2 = jnp.log(jnp.maximum(a2, 0.0) + 1.0)
    kb = jnp.dot(a2, wk_ref[...], preferred_element_type=jnp.float32) + bk_ref[...]
    vb = jnp.dot(a2, wv_ref[...], preferred_element_type=jnp.float32) + bv_ref[...]
    for b in range(PB):
        for j in range(8):
            v_ref[b, 0, pl.ds(j * NVAR, NVAR)] = (
                vb[b * NVAR:(b + 1) * NVAR, j * 128:(j + 1) * 128])
    qblk = qblk_ref[0]
    sc_halves = [
        lax.dot_general(
            qblk, kb[b * NVAR:(b + 1) * NVAR], (((1,), (1,)), ((), ())),
            preferred_element_type=jnp.float32) * SCALE
        for b in range(PB)]
    s = jnp.concatenate(sc_halves, axis=0)

    lane = lax.broadcasted_iota(jnp.int32, (PB * R, NVAR), 1)
    vals, idxs = [], []
    for _ in range(K):
        m = jnp.max(s, axis=1, keepdims=True)
        cand = jnp.where(s == m, lane, jnp.int32(NVAR))
        j = jnp.min(cand, axis=1, keepdims=True)
        vals.append(m)
        idxs.append(j)
        s = jnp.where(lane == j, -jnp.inf, s)
    topv = jnp.concatenate(vals, axis=1)
    topi = jnp.concatenate(idxs, axis=1)

    e = jnp.exp(topv - topv[:, 0:1])
    attn = e / jnp.sum(e, axis=1, keepdims=True)
    hrow2 = lax.broadcasted_iota(jnp.int32, (PB * R, K), 0) % R // G
    bsvec = bsA + SEG * (lax.broadcasted_iota(jnp.int32, (PB * R, K), 0) // R)
    val = bsvec * (NVAR * 8) + (hrow2 // 2) * NVAR + topi
    for b in range(PB):
        attn_ref[b, 0] = attn[b * R:(b + 1) * R]
        idx_ref[b, 0] = topi[b * R:(b + 1) * R]
        sp = jnp.dot(p_ref[...],
                     val[b * R:(b + 1) * R].astype(jnp.float32),
                     preferred_element_type=jnp.float32,
                     precision=lax.Precision.HIGHEST)
        sidx_ref[b, 0] = sp.astype(jnp.int32)


def _sc_kernel(sidx_hbm, v_hbm, wg_hbm, bg_hbm, xo_hbm,
               sidx_all, rows0, rows1, outb, wg_bc, bg_bc, sem0, sem1):
    c = lax.axis_index("c")
    s = lax.axis_index("s")
    wid = s * 2 + c
    pltpu.sync_copy(wg_hbm, wg_bc)
    pltpu.sync_copy(bg_hbm, bg_bc)
    bs = xo_hbm.shape[0] // R
    per_w = (bs * R) // NW
    base = wid * per_w
    nch = per_w // CB
    pch = R // CB
    pltpu.sync_copy(sidx_hbm.at[pl.ds(wid * nch, nch)], sidx_all)

    bufs = (rows0, rows1)
    sems = (sem0, sem1)
    pltpu.async_copy(v_hbm.at[sidx_all.at[0]], rows0, sem0)

    def chunk(t, ph):
        tn = jnp.minimum(t + 1, nch - 1)
        pltpu.async_copy(
            v_hbm.at[sidx_all.at[tn]], bufs[1 - ph], sems[1 - ph])
        pltpu.make_async_copy(
            v_hbm.at[sidx_all.at[t]], bufs[ph], sems[ph]).wait()
        rows = bufs[ph]
        for i in range(CB):
            ridx = base + t * CB + i
            dl = lax.rem(ridx, R)
            g = lax.div(dl, H)
            h = lax.rem(dl, H)
            col0 = lax.rem(h, 2) * DH
            wvecs = [wg_bc[g * K + r] for r in range(K)]
            accs = []
            for q in range(4):
                acc = jnp.zeros((16,), jnp.float32)
                for r in range(K):
                    acc = acc + rows[i * K + r,
                                     pl.ds(col0 + q * 16, 16)] * wvecs[r]
                accs.append(acc + bg_bc[g])
            mx4 = jnp.maximum(jnp.maximum(accs[0], accs[1]),
                              jnp.maximum(accs[2], accs[3]))
            mn4 = jnp.minimum(jnp.minimum(accs[0], accs[1]),
                              jnp.minimum(accs[2], accs[3]))
            mx = lax.reduce_max(mx4, (0,))
            mn = lax.reduce_min(mn4, (0,))
            denom = jnp.maximum(mx - mn, 1e-6)
            lr = lax.rem(t, pch) * CB + i
            for q in range(4):
                outb[lr, pl.ds(q * 16, 16)] = jnp.exp((accs[q] - mn) / denom)
        @pl.when(lax.rem(t, pch) == pch - 1)
        def _():
            bs_i = lax.div(base + t * CB, R)
            pltpu.sync_copy(outb, xo_hbm.at[pl.ds(bs_i * R, R)])

    def body(u, carry):
        chunk(2 * u, 0)
        chunk(2 * u + 1, 1)
        return carry

    lax.fori_loop(0, nch // 2, body, 0)
    pltpu.make_async_copy(
        v_hbm.at[sidx_all.at[nch - 1]], rows0, sem0).wait()


def _out_kernel(xo_ref, wo_ref, bo_ref, out_ref):
    out_ref[...] = jnp.dot(xo_ref[...], wo_ref[...],
                           preferred_element_type=jnp.float32) + bo_ref[...]


@jax.jit
def _run(x, cluster, Wq, bq, Wk, bk, Wv, bv, Wg, bg, Wo, bo):
    bs = x.shape[0]
    nb = bs // SEG

    qblk = pl.pallas_call(
        _q_kernel,
        grid=(SEG,),
        in_specs=[
            pl.BlockSpec((1, G, D), lambda s: (s, 0, 0)),
            pl.BlockSpec((D, D), lambda s: (0, 0)),
            pl.BlockSpec((1, D), lambda s: (0, 0)),
        ],
        out_specs=pl.BlockSpec((1, R, D), lambda s: (s, 0, 0)),
        out_shape=jax.ShapeDtypeStruct((SEG, R, D), jnp.float32),
    )(cluster, Wq, bq.reshape(1, D))

    dl = np.arange(R)
    pmat = np.zeros((R, R), np.float32)
    pmat[dl, (dl % H) * G + dl // H] = 1.0
    perm = jnp.asarray(pmat)
    wg_bc = jnp.broadcast_to(Wg.reshape(G * K, 1), (G * K, 16))
    bg_bc = jnp.broadcast_to(bg.reshape(G, 1), (G, 16))

    def run_chunk(xc):
        cbs = xc.shape[0]
        cnb = cbs // SEG
        x4 = xc.reshape(cnb, SEG, NVAR, D)
        v4, attn4, idx4, sidx4 = pl.pallas_call(
            _tc1_kernel,
            grid=(SEG, cnb // PB),
            in_specs=[
                pl.BlockSpec((PB, 1, NVAR, D), lambda s, b: (b, s, 0, 0)),
                pl.BlockSpec((1, R, D), lambda s, b: (s, 0, 0)),
                pl.BlockSpec((D, D), lambda s, b: (0, 0)),
                pl.BlockSpec((1, D), lambda s, b: (0, 0)),
                pl.BlockSpec((D, D), lambda s, b: (0, 0)),
                pl.BlockSpec((1, D), lambda s, b: (0, 0)),
                pl.BlockSpec((R, R), lambda s, b: (0, 0)),
            ],
            out_specs=[
                pl.BlockSpec((PB, 1, NVAR * 8, 128), lambda s, b: (b, s, 0, 0)),
                pl.BlockSpec((PB, 1, R, K), lambda s, b: (b, s, 0, 0)),
                pl.BlockSpec((PB, 1, R, K), lambda s, b: (b, s, 0, 0)),
                pl.BlockSpec((PB, 1, R, K), lambda s, b: (b, s, 0, 0)),
            ],
            out_shape=[
                jax.ShapeDtypeStruct((cnb, SEG, NVAR * 8, 128), jnp.float32),
                jax.ShapeDtypeStruct((cnb, SEG, R, K), jnp.float32),
                jax.ShapeDtypeStruct((cnb, SEG, R, K), jnp.int32),
                jax.ShapeDtypeStruct((cnb, SEG, R, K), jnp.int32),
            ],
        )(x4, qblk, Wk, bk.reshape(1, D), Wv, bv.reshape(1, D), perm)
        v = v4.reshape(cbs * NVAR * 8, 128)
        attn_t = attn4.reshape(cbs, R, K)
        idx_t = idx4.reshape(cbs, R, K)
        sidx_t = sidx4.reshape(cbs, R, K)

        sc = pl.kernel(
            _sc_kernel,
            out_type=jax.ShapeDtypeStruct((cbs * R, DH), jnp.float32),
            mesh=plsc.VectorSubcoreMesh(core_axis_name="c",
                                        subcore_axis_name="s",
                                        num_cores=2, num_subcores=16),
            compiler_params=pltpu.CompilerParams(needs_layout_passes=False),
            scratch_types=[
                pltpu.VMEM(((cbs * R // NW) // CB, CB * K), jnp.int32),
                pltpu.VMEM((CB * K, 128), jnp.float32),
                pltpu.VMEM((CB * K, 128), jnp.float32),
                pltpu.VMEM((R, DH), jnp.float32),
                pltpu.VMEM((G * K, 16), jnp.float32),
                pltpu.VMEM((G, 16), jnp.float32),
                pltpu.SemaphoreType.DMA,
                pltpu.SemaphoreType.DMA,
            ],
        )
        xo = sc(sidx_t.reshape(cbs * R * K // (CB * K), CB * K),
                v, wg_bc, bg_bc)

        xo_mat = xo.reshape(cbs * G, D)
        rb = 256 if (cbs * G) % 256 == 0 else cbs * G
        out = pl.pallas_call(
            _out_kernel,
            grid=((cbs * G) // rb,),
            in_specs=[
                pl.BlockSpec((rb, D), lambda i: (i, 0)),
                pl.BlockSpec((D, D), lambda i: (0, 0)),
                pl.BlockSpec((1, D), lambda i: (0, 0)),
            ],
            out_specs=pl.BlockSpec((rb, D), lambda i: (i, 0)),
            out_shape=jax.ShapeDtypeStruct((cbs * G, D), jnp.float32),
        )(xo_mat, Wo, bo.reshape(1, D))
        return out.reshape(cbs, G, D), attn_t, idx_t

    nchunks = 2 if bs % 256 == 0 else 1
    if nchunks == 2:
        h = bs // 2
        o1, a1, i1 = run_chunk(x[:h])
        o2, a2, i2 = run_chunk(x[h:])
        out = jnp.concatenate([o1, o2], axis=0)
        attn_t = jnp.concatenate([a1, a2], axis=0)
        idx_t = jnp.concatenate([i1, i2], axis=0)
    else:
        out, attn_t, idx_t = run_chunk(x)

    return (out,
            attn_t.reshape(bs, H, G, K),
            idx_t.reshape(bs, H, G, K))


def kernel(x, cluster, Wq, bq, Wk, bk, Wv, bv, Wg, bg, Wo, bo):
    return _run(x, cluster, Wq, bq, Wk, bk, Wv, bv, Wg, bg, Wo, bo)


# speedup vs baseline: 1.6632x; 1.0707x over previous
import functools
import numpy as np
import jax
import jax.numpy as jnp
from jax import lax
from jax.experimental import pallas as pl
from jax.experimental.pallas import tpu as pltpu
from jax.experimental.pallas import tpu_sc as plsc

SEG = 8
NVAR = 100
H = 16
D = 1024
G = 10
K = 10
DH = D // H
R = H * G
SCALE = float((D / H) ** -0.5)

NW = 32
CB = 8
PB = 8


def _q_kernel(cl_ref, wq_ref, bq_ref, qblk_ref):
    qq = jnp.dot(cl_ref[0], wq_ref[...], preferred_element_type=jnp.float32)
    qq = qq + bq_ref[...]
    q3 = jnp.broadcast_to(qq[None, :, :], (H, G, D))
    h_iota = lax.broadcasted_iota(jnp.int32, (H, G, D), 0)
    d_iota = lax.broadcasted_iota(jnp.int32, (H, G, D), 2)
    qblk_ref[0] = jnp.where(d_iota // DH == h_iota, q3, 0.0).reshape(R, D)


def _tc1_kernel(x_ref, qblk_ref, wk_ref, bk_ref, wv_ref, bv_ref, p_ref,
                v_ref, attn_ref, idx_ref, sidx_ref):
    bsA = PB * pl.program_id(1) * SEG + pl.program_id(0)
    a2 = x_ref[:, 0].reshape(PB * NVAR, D)
    a2 = jnp.log(jnp.maximum(a2, 0.0) + 1.0)
    kb = jnp.dot(a2, wk_ref[...], preferred_element_type=jnp.float32) + bk_ref[...]
    vb = jnp.dot(a2, wv_ref[...], preferred_element_type=jnp.float32) + bv_ref[...]
    for b in range(PB):
        for j in range(8):
            v_ref[b, 0, pl.ds(j * NVAR, NVAR)] = (
                vb[b * NVAR:(b + 1) * NVAR, j * 128:(j + 1) * 128])
    qblk = qblk_ref[0]
    sc_halves = [
        lax.dot_general(
            qblk, kb[b * NVAR:(b + 1) * NVAR], (((1,), (1,)), ((), ())),
            preferred_element_type=jnp.float32) * SCALE
        for b in range(PB)]
    s = jnp.concatenate(sc_halves, axis=0)

    lane = lax.broadcasted_iota(jnp.int32, (PB * R, NVAR), 1)
    vals, idxs = [], []
    for _ in range(K):
        m = jnp.max(s, axis=1, keepdims=True)
        cand = jnp.where(s == m, lane, jnp.int32(NVAR))
        j = jnp.min(cand, axis=1, keepdims=True)
        vals.append(m)
        idxs.append(j)
        s = jnp.where(lane == j, -jnp.inf, s)
    topv = jnp.concatenate(vals, axis=1)
    topi = jnp.concatenate(idxs, axis=1)

    e = jnp.exp(topv - topv[:, 0:1])
    attn = e / jnp.sum(e, axis=1, keepdims=True)
    hrow2 = lax.broadcasted_iota(jnp.int32, (PB * R, K), 0) % R // G
    bsvec = bsA + SEG * (lax.broadcasted_iota(jnp.int32, (PB * R, K), 0) // R)
    val = bsvec * (NVAR * 8) + (hrow2 // 2) * NVAR + topi
    for b in range(PB):
        attn_ref[b, 0] = attn[b * R:(b + 1) * R]
        idx_ref[b, 0] = topi[b * R:(b + 1) * R]
        sp = jnp.dot(p_ref[...],
                     val[b * R:(b + 1) * R].astype(jnp.float32),
                     preferred_element_type=jnp.float32,
                     precision=lax.Precision.HIGHEST)
        sidx_ref[b, 0] = sp.astype(jnp.int32)


def _sc_kernel(sidx_hbm, v_hbm, wg_hbm, bg_hbm, xo_hbm,
               sidx_all, rows0, rows1, outb, wg_bc, bg_bc, sem0, sem1):
    c = lax.axis_index("c")
    s = lax.axis_index("s")
    wid = s * 2 + c
    pltpu.sync_copy(wg_hbm, wg_bc)
    pltpu.sync_copy(bg_hbm, bg_bc)
    bs = xo_hbm.shape[0] // R
    per_w = (bs * R) // NW
    base = wid * per_w
    nch = per_w // CB
    pch = R // CB
    pltpu.sync_copy(sidx_hbm.at[pl.ds(wid * nch, nch)], sidx_all)

    bufs = (rows0, rows1)
    sems = (sem0, sem1)
    pltpu.async_copy(v_hbm.at[sidx_all.at[0]], rows0, sem0)

    def chunk(t, ph):
        tn = jnp.minimum(t + 1, nch - 1)
        pltpu.async_copy(
            v_hbm.at[sidx_all.at[tn]], bufs[1 - ph], sems[1 - ph])
        pltpu.make_async_copy(
            v_hbm.at[sidx_all.at[t]], bufs[ph], sems[ph]).wait()
        rows = bufs[ph]
        for i in range(CB):
            ridx = base + t * CB + i
            dl = lax.rem(ridx, R)
            g = lax.div(dl, H)
            h = lax.rem(dl, H)
            col0 = lax.rem(h, 2) * DH
            wvecs = [wg_bc[g * K + r] for r in range(K)]
            accs = []
            for q in range(4):
                acc = jnp.zeros((16,), jnp.float32)
                for r in range(K):
                    acc = acc + rows[i * K + r,
                                     pl.ds(col0 + q * 16, 16)] * wvecs[r]
                accs.append(acc + bg_bc[g])
            mx4 = jnp.maximum(jnp.maximum(accs[0], accs[1]),
                              jnp.maximum(accs[2], accs[3]))
            mn4 = jnp.minimum(jnp.minimum(accs[0], accs[1]),
                              jnp.minimum(accs[2], accs[3]))
            mx = lax.reduce_max(mx4, (0,))
            mn = lax.reduce_min(mn4, (0,))
            denom = jnp.maximum(mx - mn, 1e-6)
            lr = lax.rem(t, pch) * CB + i
            for q in range(4):
                outb[lr, pl.ds(q * 16, 16)] = jnp.exp((accs[q] - mn) / denom)
        @pl.when(lax.rem(t, pch) == pch - 1)
        def _():
            bs_i = lax.div(base + t * CB, R)
            pltpu.sync_copy(outb, xo_hbm.at[pl.ds(bs_i * R, R)])

    def body(u, carry):
        chunk(2 * u, 0)
        chunk(2 * u + 1, 1)
        return carry

    lax.fori_loop(0, nch // 2, body, 0)
    pltpu.make_async_copy(
        v_hbm.at[sidx_all.at[nch - 1]], rows0, sem0).wait()


def _out_kernel(xo_ref, wo_ref, bo_ref, out_ref):
    out_ref[...] = jnp.dot(xo_ref[...], wo_ref[...],
                           preferred_element_type=jnp.float32) + bo_ref[...]


@jax.jit
def _run(x, cluster, Wq, bq, Wk, bk, Wv, bv, Wg, bg, Wo, bo):
    bs = x.shape[0]
    nb = bs // SEG

    qblk = pl.pallas_call(
        _q_kernel,
        grid=(SEG,),
        in_specs=[
            pl.BlockSpec((1, G, D), lambda s: (s, 0, 0)),
            pl.BlockSpec((D, D), lambda s: (0, 0)),
            pl.BlockSpec((1, D), lambda s: (0, 0)),
        ],
        out_specs=pl.BlockSpec((1, R, D), lambda s: (s, 0, 0)),
        out_shape=jax.ShapeDtypeStruct((SEG, R, D), jnp.float32),
    )(cluster, Wq, bq.reshape(1, D))

    dl = np.arange(R)
    pmat = np.zeros((R, R), np.float32)
    pmat[dl, (dl % H) * G + dl // H] = 1.0
    perm = jnp.asarray(pmat)
    wg_bc = jnp.broadcast_to(Wg.reshape(G * K, 1), (G * K, 16))
    bg_bc = jnp.broadcast_to(bg.reshape(G, 1), (G, 16))

    def run_chunk(xc):
        cbs = xc.shape[0]
        cnb = cbs // SEG
        x4 = xc.reshape(cnb, SEG, NVAR, D)
        v4, attn4, idx4, sidx4 = pl.pallas_call(
            _tc1_kernel,
            grid=(SEG, cnb // PB),
            in_specs=[
                pl.BlockSpec((PB, 1, NVAR, D), lambda s, b: (b, s, 0, 0)),
                pl.BlockSpec((1, R, D), lambda s, b: (s, 0, 0)),
                pl.BlockSpec((D, D), lambda s, b: (0, 0)),
                pl.BlockSpec((1, D), lambda s, b: (0, 0)),
                pl.BlockSpec((D, D), lambda s, b: (0, 0)),
                pl.BlockSpec((1, D), lambda s, b: (0, 0)),
                pl.BlockSpec((R, R), lambda s, b: (0, 0)),
            ],
            out_specs=[
                pl.BlockSpec((PB, 1, NVAR * 8, 128), lambda s, b: (b, s, 0, 0)),
                pl.BlockSpec((PB, 1, R, K), lambda s, b: (b, s, 0, 0)),
                pl.BlockSpec((PB, 1, R, K), lambda s, b: (b, s, 0, 0)),
                pl.BlockSpec((PB, 1, R, K), lambda s, b: (b, s, 0, 0)),
            ],
            out_shape=[
                jax.ShapeDtypeStruct((cnb, SEG, NVAR * 8, 128), jnp.float32),
                jax.ShapeDtypeStruct((cnb, SEG, R, K), jnp.float32),
                jax.ShapeDtypeStruct((cnb, SEG, R, K), jnp.int32),
                jax.ShapeDtypeStruct((cnb, SEG, R, K), jnp.int32),
            ],
        )(x4, qblk, Wk, bk.reshape(1, D), Wv, bv.reshape(1, D), perm)
        v = v4.reshape(cbs * NVAR * 8, 128)
        attn_t = attn4.reshape(cbs, R, K)
        idx_t = idx4.reshape(cbs, R, K)
        sidx_t = sidx4.reshape(cbs, R, K)

        sc = pl.kernel(
            _sc_kernel,
            out_type=jax.ShapeDtypeStruct((cbs * R, DH), jnp.float32),
            mesh=plsc.VectorSubcoreMesh(core_axis_name="c",
                                        subcore_axis_name="s",
                                        num_cores=2, num_subcores=16),
            compiler_params=pltpu.CompilerParams(needs_layout_passes=False),
            scratch_types=[
                pltpu.VMEM(((cbs * R // NW) // CB, CB * K), jnp.int32),
                pltpu.VMEM((CB * K, 128), jnp.float32),
                pltpu.VMEM((CB * K, 128), jnp.float32),
                pltpu.VMEM((R, DH), jnp.float32),
                pltpu.VMEM((G * K, 16), jnp.float32),
                pltpu.VMEM((G, 16), jnp.float32),
                pltpu.SemaphoreType.DMA,
                pltpu.SemaphoreType.DMA,
            ],
        )
        xo = sc(sidx_t.reshape(cbs * R * K // (CB * K), CB * K),
                v, wg_bc, bg_bc)

        xo_mat = xo.reshape(cbs * G, D)
        rb = 256 if (cbs * G) % 256 == 0 else cbs * G
        out = pl.pallas_call(
            _out_kernel,
            grid=((cbs * G) // rb,),
            in_specs=[
                pl.BlockSpec((rb, D), lambda i: (i, 0)),
                pl.BlockSpec((D, D), lambda i: (0, 0)),
                pl.BlockSpec((1, D), lambda i: (0, 0)),
            ],
            out_specs=pl.BlockSpec((rb, D), lambda i: (i, 0)),
            out_shape=jax.ShapeDtypeStruct((cbs * G, D), jnp.float32),
        )(xo_mat, Wo, bo.reshape(1, D))
        return out.reshape(cbs, G, D), attn_t, idx_t

    nchunks = 2 if bs % 256 == 0 else 1
    if nchunks == 2:
        h = bs // 2
        o1, a1, i1 = run_chunk(x[:h])
        o2, a2, i2 = run_chunk(x[h:])
        out = jnp.concatenate([o1, o2], axis=0)
        attn_t = jnp.concatenate([a1, a2], axis=0)
        idx_t = jnp.concatenate([i1, i2], axis=0)
    else:
        out, attn_t, idx_t = run_chunk(x)

    return (out,
            attn_t.reshape(bs, H, G, K),
            idx_t.reshape(bs, H, G, K))


def kernel(x, cluster, Wq, bq, Wk, bk, Wv, bv, Wg, bg, Wo, bo):
    return _run(x, cluster, Wq, bq, Wk, bk, Wv, bv, Wg, bg, Wo, bo)
